# baseline (speedup 1.0000x reference)
import functools

import jax
import jax.numpy as jnp
from jax import lax
from jax.experimental import pallas as pl
from jax.experimental.pallas import tpu as pltpu
from jax.experimental.pallas import tpu_sc as plsc

_LOG2_IFUS = 10


@functools.lru_cache(maxsize=None)
def _build(n_obs: int, table_n: int):
    info = plsc.get_sparse_core_info()
    nc, ns, nl = info.num_cores, info.num_subcores, info.num_lanes
    nw = nc * ns
    per_w = n_obs // nw
    chunk = 8192
    n_chunks = per_w // chunk
    n_pairs = n_chunks // 2
    seg = table_n // ns
    mesh = plsc.VectorSubcoreMesh(core_axis_name="c", subcore_axis_name="s")

    @functools.partial(
        pl.kernel,
        out_type=jax.ShapeDtypeStruct((n_obs,), jnp.float32),
        mesh=mesh,
        scratch_types=[
            pltpu.VMEM_SHARED((table_n,), jnp.float32),
            pltpu.VMEM((chunk,), jnp.int32),
            pltpu.VMEM((chunk,), jnp.int32),
            pltpu.VMEM((chunk,), jnp.int32),
            pltpu.VMEM((chunk,), jnp.int32),
            pltpu.VMEM((chunk,), jnp.int32),
            pltpu.VMEM((chunk,), jnp.int32),
            pltpu.VMEM((chunk,), jnp.float32),
            pltpu.VMEM((chunk,), jnp.float32),
            pltpu.SemaphoreType.DMA,
            pltpu.SemaphoreType.DMA,
            pltpu.SemaphoreType.DMA,
            pltpu.SemaphoreType.DMA,
            pltpu.SemaphoreType.DMA,
            pltpu.SemaphoreType.DMA,
        ],
    )
    def gather_kernel(tile_hbm, ifu_hbm, table_hbm, out_hbm,
                      table_sh, t0, t1, i0, i1, f0, f1, v0, v1,
                      sin0, sin1, sg0, sg1, so0, so1):
        sid = lax.axis_index("s")
        cid = lax.axis_index("c")
        wid = sid * nc + cid
        base = wid * per_w

        t_v, i_v, f_v, vals_v = (t0, t1), (i0, i1), (f0, f1), (v0, v1)
        sem_in, sem_g, sem_out = (sin0, sin1), (sg0, sg1), (so0, so1)

        def off(g):
            return pl.multiple_of(base + g * chunk, chunk)

        def start_in(g, b):
            pltpu.async_copy(tile_hbm.at[pl.ds(off(g), chunk)], t_v[b], sem_in[b])
            pltpu.async_copy(ifu_hbm.at[pl.ds(off(g), chunk)], i_v[b], sem_in[b])

        def wait_in(g, b):
            pltpu.make_async_copy(tile_hbm.at[pl.ds(off(g), chunk)], t_v[b], sem_in[b]).wait()
            pltpu.make_async_copy(ifu_hbm.at[pl.ds(off(g), chunk)], i_v[b], sem_in[b]).wait()

        def compute_flat(b):
            def vec_body(j, c):
                s = pl.ds(pl.multiple_of(j * nl, nl), nl)
                f_v[b][s] = (t_v[b][s] << _LOG2_IFUS) | i_v[b][s]
                return c
            lax.fori_loop(0, chunk // nl, vec_body, 0, unroll=8)

        def start_gather(g, b):
            pltpu.async_copy(table_sh.at[f_v[b]], vals_v[b], sem_g[b])

        def wait_gather(b):
            pltpu.make_async_copy(table_sh.at[f_v[b]], vals_v[b], sem_g[b]).wait()

        def start_out(g, b):
            pltpu.async_copy(vals_v[b], out_hbm.at[pl.ds(off(g), chunk)], sem_out[b])

        def wait_out(g, b):
            pltpu.make_async_copy(vals_v[b], out_hbm.at[pl.ds(off(g), chunk)], sem_out[b]).wait()

        start_in(0, 0)
        pltpu.sync_copy(table_hbm.at[pl.ds(sid * seg, seg)],
                        table_sh.at[pl.ds(sid * seg, seg)])
        plsc.subcore_barrier()

        def slot(g, b):
            @pl.when(g + 1 < n_chunks)
            def _():
                start_in(g + 1, 1 - b)
            wait_in(g, b)
            compute_flat(b)

            @pl.when(g >= 2)
            def _():
                wait_out(g - 2, b)
            start_gather(g, b)

            @pl.when(g >= 1)
            def _():
                wait_gather(1 - b)
                start_out(g - 1, 1 - b)

        def pair_body(p, carry):
            slot(2 * p, 0)
            slot(2 * p + 1, 1)
            return carry

        lax.fori_loop(0, n_pairs, pair_body, 0)

        g_last = n_chunks - 1
        wait_gather(1)
        start_out(g_last, 1)
        wait_out(g_last - 1, 0)
        wait_out(g_last, 1)

    return gather_kernel


def kernel(tile_idx, ifu_idx, ifu_values):
    n_obs = tile_idx.shape[0]
    n_tiles, n_ifus = ifu_values.shape
    table = ifu_values.reshape(n_tiles * n_ifus)
    fn = _build(n_obs, n_tiles * n_ifus)
    return fn(tile_idx.astype(jnp.int32), ifu_idx.astype(jnp.int32), table)


# speedup vs baseline: 1.0038x; 1.0038x over previous
import functools

import jax
import jax.numpy as jnp
from jax import lax
from jax.experimental import pallas as pl
from jax.experimental.pallas import tpu as pltpu
from jax.experimental.pallas import tpu_sc as plsc

_LOG2_IFUS = 10


@functools.lru_cache(maxsize=None)
def _build(n_obs: int, table_n: int):
    info = plsc.get_sparse_core_info()
    nc, ns, nl = info.num_cores, info.num_subcores, info.num_lanes
    nw = nc * ns
    per_w = n_obs // nw
    chunk = 4096
    n_chunks = per_w // chunk
    n_pairs = n_chunks // 2
    seg = table_n // ns
    mesh = plsc.VectorSubcoreMesh(core_axis_name="c", subcore_axis_name="s")

    @functools.partial(
        pl.kernel,
        out_type=jax.ShapeDtypeStruct((n_obs,), jnp.float32),
        mesh=mesh,
        scratch_types=[
            pltpu.VMEM_SHARED((table_n,), jnp.float32),
            pltpu.VMEM((chunk,), jnp.int32),
            pltpu.VMEM((chunk,), jnp.int32),
            pltpu.VMEM((chunk,), jnp.int32),
            pltpu.VMEM((chunk,), jnp.int32),
            pltpu.VMEM((chunk,), jnp.int32),
            pltpu.VMEM((chunk,), jnp.int32),
            pltpu.VMEM((chunk,), jnp.float32),
            pltpu.VMEM((chunk,), jnp.float32),
            pltpu.SemaphoreType.DMA,
            pltpu.SemaphoreType.DMA,
            pltpu.SemaphoreType.DMA,
            pltpu.SemaphoreType.DMA,
            pltpu.SemaphoreType.DMA,
            pltpu.SemaphoreType.DMA,
        ],
    )
    def gather_kernel(tile_hbm, ifu_hbm, table_hbm, out_hbm,
                      table_sh, t0, t1, i0, i1, f0, f1, v0, v1,
                      sin0, sin1, sg0, sg1, so0, so1):
        sid = lax.axis_index("s")
        cid = lax.axis_index("c")
        wid = sid * nc + cid
        base = wid * per_w

        t_v, i_v, f_v, vals_v = (t0, t1), (i0, i1), (f0, f1), (v0, v1)
        sem_in, sem_g, sem_out = (sin0, sin1), (sg0, sg1), (so0, so1)

        def off(g):
            return pl.multiple_of(base + g * chunk, chunk)

        def start_in(g, b):
            pltpu.async_copy(tile_hbm.at[pl.ds(off(g), chunk)], t_v[b], sem_in[b])
            pltpu.async_copy(ifu_hbm.at[pl.ds(off(g), chunk)], i_v[b], sem_in[b])

        def wait_in(g, b):
            pltpu.make_async_copy(tile_hbm.at[pl.ds(off(g), chunk)], t_v[b], sem_in[b]).wait()
            pltpu.make_async_copy(ifu_hbm.at[pl.ds(off(g), chunk)], i_v[b], sem_in[b]).wait()

        def compute_flat(b):
            def vec_body(j, c):
                s = pl.ds(pl.multiple_of(j * nl, nl), nl)
                f_v[b][s] = (t_v[b][s] << _LOG2_IFUS) | i_v[b][s]
                return c
            lax.fori_loop(0, chunk // nl, vec_body, 0, unroll=8)

        def start_gather(g, b):
            pltpu.async_copy(table_sh.at[f_v[b]], vals_v[b], sem_g[b])

        def wait_gather(b):
            pltpu.make_async_copy(table_sh.at[f_v[b]], vals_v[b], sem_g[b]).wait()

        def start_out(g, b):
            pltpu.async_copy(vals_v[b], out_hbm.at[pl.ds(off(g), chunk)], sem_out[b])

        def wait_out(g, b):
            pltpu.make_async_copy(vals_v[b], out_hbm.at[pl.ds(off(g), chunk)], sem_out[b]).wait()

        start_in(0, 0)
        pltpu.sync_copy(table_hbm.at[pl.ds(sid * seg, seg)],
                        table_sh.at[pl.ds(sid * seg, seg)])
        plsc.subcore_barrier()

        def slot(g, b):
            @pl.when(g + 1 < n_chunks)
            def _():
                start_in(g + 1, 1 - b)
            wait_in(g, b)
            compute_flat(b)

            @pl.when(g >= 2)
            def _():
                wait_out(g - 2, b)
            start_gather(g, b)

            @pl.when(g >= 1)
            def _():
                wait_gather(1 - b)
                start_out(g - 1, 1 - b)

        def pair_body(p, carry):
            slot(2 * p, 0)
            slot(2 * p + 1, 1)
            return carry

        lax.fori_loop(0, n_pairs, pair_body, 0)

        g_last = n_chunks - 1
        wait_gather(1)
        start_out(g_last, 1)
        wait_out(g_last - 1, 0)
        wait_out(g_last, 1)

    return gather_kernel


def kernel(tile_idx, ifu_idx, ifu_values):
    n_obs = tile_idx.shape[0]
    n_tiles, n_ifus = ifu_values.shape
    table = ifu_values.reshape(n_tiles * n_ifus)
    fn = _build(n_obs, n_tiles * n_ifus)
    return fn(tile_idx.astype(jnp.int32), ifu_idx.astype(jnp.int32), table)
